# baseline (speedup 1.0000x reference)
import functools

import jax
import jax.numpy as jnp
from jax import lax
from jax.experimental import pallas as pl
from jax.experimental.pallas import tpu as pltpu
from jax.experimental.pallas import tpu_sc as plsc

_INFO = plsc.get_sparse_core_info()
_NC, _NS, _L = _INFO.num_cores, _INFO.num_subcores, _INFO.num_lanes
_NW = _NC * _NS
_NQ = 4


def _make_sc_partials(B, V, S):
    rpt = B // _NW
    sizes = (25088, 25088, 25088, 24704)
    offs = (0, 25088, 50176, 75264)
    tail0 = offs[-1] + sizes[-1]
    tailn = V - tail0
    mesh = plsc.VectorSubcoreMesh(core_axis_name="c", subcore_axis_name="s")

    @functools.partial(
        pl.kernel,
        out_type=(
            jax.ShapeDtypeStruct((B,), jnp.float32),
            jax.ShapeDtypeStruct((B,), jnp.float32),
        ),
        mesh=mesh,
        compiler_params=pltpu.CompilerParams(needs_layout_passes=False),
        scratch_types=[
            pltpu.VMEM((rpt,), jnp.int32),
            pltpu.VMEM((S,), jnp.int32),
            pltpu.VMEM((S,), jnp.float32),
            pltpu.VMEM((sizes[0],), jnp.float32),
            pltpu.VMEM((sizes[1],), jnp.float32),
            pltpu.VMEM((sizes[2],), jnp.float32),
            pltpu.VMEM((sizes[3],), jnp.float32),
            pltpu.VMEM((tailn,), jnp.float32),
            pltpu.VMEM((rpt,), jnp.float32),
            pltpu.VMEM((rpt,), jnp.float32),
            pltpu.SemaphoreType.DMA,
            pltpu.SemaphoreType.DMA,
            pltpu.SemaphoreType.DMA,
            pltpu.SemaphoreType.DMA,
        ],
    )
    def sc_partials(x_hbm, tail_hbm, tgt_hbm, smp_hbm, outA_hbm, outP_hbm,
                    tgi_v, sidx_v, s_v, q0_v, q1_v, q2_v, q3_v, tl_v,
                    oA_v, oP_v, sem0, sem1, sem2, sem3):
        wid = lax.axis_index("s") * _NC + lax.axis_index("c")
        base = wid * rpt
        lane0 = lax.iota(jnp.int32, _L) == 0
        qbufs = (q0_v, q1_v, q2_v, q3_v)
        qsems = (sem0, sem1, sem2, sem3)
        neginf = jnp.full((_L,), -jnp.inf, jnp.float32)

        def fire(r, k):
            pltpu.async_copy(x_hbm.at[base + r, pl.ds(offs[k], sizes[k])],
                             qbufs[k], qsems[k])

        for k in range(_NQ):
            fire(0, k)

        def row_step(r, carry):
            pltpu.sync_copy(smp_hbm.at[base + r], sidx_v)
            pltpu.sync_copy(tail_hbm.at[base + r], tl_v)
            rvec = jnp.full((_L,), r, jnp.int32)
            tidx = plsc.load_gather(tgi_v, [rvec])
            nxt = jnp.minimum(r + 1, rpt - 1)

            tvec = jnp.zeros((_L,), jnp.float32)
            mvec = neginf
            for k in range(_NQ + 1):
                if k < _NQ:
                    c0, cn, q_v = offs[k], sizes[k], qbufs[k]
                    pltpu.make_async_copy(
                        x_hbm.at[base + r, pl.ds(c0, cn)], q_v,
                        qsems[k]).wait()
                else:
                    c0, cn, q_v = tail0, tailn, tl_v

                def qpass(j, mv):
                    sl = pl.ds(j * _L, _L)
                    idx = sidx_v[sl]
                    local = idx - c0
                    inq = (idx >= c0) & (idx < c0 + cn)
                    v = plsc.load_gather(q_v, [jnp.where(inq, local, 0)])
                    s_v[sl] = jnp.where(inq, v, s_v[sl])
                    return jnp.maximum(mv, jnp.where(inq, v, neginf))
                mvec = lax.fori_loop(0, S // _L, qpass, mvec)

                tloc = tidx - c0
                t_inq = (tidx >= c0) & (tidx < c0 + cn)
                tv = plsc.load_gather(q_v, [jnp.where(t_inq, tloc, 0)])
                tvec = jnp.where(t_inq, tv, tvec)

                if k < _NQ:
                    fire(nxt, k)

            m = lax.reduce_max(mvec, (0,))
            zero = jnp.zeros((_L,), jnp.float32)

            def p2(j, acc):
                accE, accA, accP = acc
                v = s_v[pl.ds(j * _L, _L)]
                e = jnp.exp(v - m)
                sig = 1.0 / (1.0 + jnp.exp(v - tvec))
                return (accE + e, accA + e * sig, accP + e * v * v)
            accE, accA, accP = lax.fori_loop(0, S // _L, p2,
                                             (zero, zero, zero))

            E = lax.reduce_sum(accE, (0,))
            A = lax.reduce_sum(accA, (0,))
            P = lax.reduce_sum(accP, (0,))
            Evec = jnp.full((_L,), E)
            plsc.store_scatter(oA_v, [rvec], jnp.full((_L,), A) / Evec,
                               mask=lane0)
            plsc.store_scatter(oP_v, [rvec], jnp.full((_L,), P) / Evec,
                               mask=lane0)
            return carry

        pltpu.sync_copy(tgt_hbm.at[pl.ds(base, rpt)], tgi_v)
        lax.fori_loop(0, rpt, row_step, 0)
        for k in range(_NQ):
            pltpu.make_async_copy(
                x_hbm.at[base + rpt - 1, pl.ds(offs[k], sizes[k])],
                qbufs[k], qsems[k]).wait()

        pltpu.sync_copy(oA_v, outA_hbm.at[pl.ds(base, rpt)])
        pltpu.sync_copy(oP_v, outP_hbm.at[pl.ds(base, rpt)])

    return sc_partials


def _finish(a, p):
    B = a.shape[0]

    def body(a_ref, p_ref, o_ref):
        o_ref[0, 0] = jnp.mean(-jnp.log(a_ref[...]) + p_ref[...])

    out = pl.pallas_call(
        body,
        out_shape=jax.ShapeDtypeStruct((1, 1), jnp.float32),
        out_specs=pl.BlockSpec(memory_space=pltpu.SMEM),
    )(a.reshape(8, B // 8), p.reshape(8, B // 8))
    return out[0, 0]


def kernel(input, target, samples):
    B, V = input.shape
    S = samples.shape[1]
    tgt = target.astype(jnp.int32)
    smp = samples.astype(jnp.int32)
    tail = input[:, 99968:]
    outA, outP = _make_sc_partials(B, V, S)(input, tail, tgt, smp)
    return _finish(outA, outP)


# speedup vs baseline: 1.0167x; 1.0167x over previous
import functools

import jax
import jax.numpy as jnp
from jax import lax
from jax.experimental import pallas as pl
from jax.experimental.pallas import tpu as pltpu
from jax.experimental.pallas import tpu_sc as plsc

_INFO = plsc.get_sparse_core_info()
_NC, _NS, _L = _INFO.num_cores, _INFO.num_subcores, _INFO.num_lanes
_NW = _NC * _NS
_NQ = 4


def _make_sc_partials(B, V, S):
    rpt = B // _NW
    sizes = (50048, 49920)
    offs = (0, 50048)
    tail0 = offs[-1] + sizes[-1]
    tailn = V - tail0
    mesh = plsc.VectorSubcoreMesh(core_axis_name="c", subcore_axis_name="s")

    @functools.partial(
        pl.kernel,
        out_type=(
            jax.ShapeDtypeStruct((B,), jnp.float32),
            jax.ShapeDtypeStruct((B,), jnp.float32),
        ),
        mesh=mesh,
        compiler_params=pltpu.CompilerParams(needs_layout_passes=False),
        scratch_types=[
            pltpu.VMEM((rpt,), jnp.int32),
            pltpu.VMEM((S,), jnp.int32),
            pltpu.VMEM((S,), jnp.float32),
            pltpu.VMEM((sizes[0],), jnp.float32),
            pltpu.VMEM((sizes[1],), jnp.float32),
            pltpu.VMEM((tailn,), jnp.float32),
            pltpu.VMEM((rpt,), jnp.float32),
            pltpu.VMEM((rpt,), jnp.float32),
            pltpu.SemaphoreType.DMA,
            pltpu.SemaphoreType.DMA,
        ],
    )
    def sc_partials(x_hbm, tail_hbm, tgt_hbm, smp_hbm, outA_hbm, outP_hbm,
                    tgi_v, sidx_v, s_v, q0_v, q1_v, tl_v,
                    oA_v, oP_v, sem0, sem1):
        wid = lax.axis_index("s") * _NC + lax.axis_index("c")
        base = wid * rpt
        lane0 = lax.iota(jnp.int32, _L) == 0
        qbufs = (q0_v, q1_v)
        qsems = (sem0, sem1)
        neginf = jnp.full((_L,), -jnp.inf, jnp.float32)

        def fire(r, k):
            pltpu.async_copy(x_hbm.at[base + r, pl.ds(offs[k], sizes[k])],
                             qbufs[k], qsems[k])

        for k in range(2):
            fire(0, k)

        def row_step(r, carry):
            pltpu.sync_copy(smp_hbm.at[base + r], sidx_v)
            pltpu.sync_copy(tail_hbm.at[base + r], tl_v)
            rvec = jnp.full((_L,), r, jnp.int32)
            tidx = plsc.load_gather(tgi_v, [rvec])
            nxt = jnp.minimum(r + 1, rpt - 1)

            pltpu.make_async_copy(
                x_hbm.at[base + r, pl.ds(0, sizes[0])], q0_v, sem0).wait()

            def qpass0(j, mv):
                sl = pl.ds(j * _L, _L)
                idx = sidx_v[sl]
                inq = idx < sizes[0]
                v = plsc.load_gather(q0_v, [jnp.where(inq, idx, 0)])
                s_v[sl] = jnp.where(inq, v, s_v[sl])
                return jnp.maximum(mv, jnp.where(inq, v, neginf))
            mvec = lax.fori_loop(0, S // _L, qpass0, neginf)

            t_inq = tidx < sizes[0]
            tv = plsc.load_gather(q0_v, [jnp.where(t_inq, tidx, 0)])
            tvec = jnp.where(t_inq, tv, jnp.zeros((_L,), jnp.float32))
            fire(nxt, 0)

            c0 = offs[1]
            pltpu.make_async_copy(
                x_hbm.at[base + r, pl.ds(c0, sizes[1])], q1_v, sem1).wait()

            def qpass1(j, mv):
                sl = pl.ds(j * _L, _L)
                idx = sidx_v[sl]
                in_t = idx >= tail0
                in_m = (idx >= c0) & (idx < tail0)
                vm = plsc.load_gather(q1_v, [jnp.where(in_m, idx - c0, 0)])
                vt = plsc.load_gather(tl_v, [jnp.where(in_t, idx - tail0, 0)])
                v = jnp.where(in_t, vt, vm)
                inq = idx >= c0
                s_v[sl] = jnp.where(inq, v, s_v[sl])
                return jnp.maximum(mv, jnp.where(inq, v, neginf))
            mvec = lax.fori_loop(0, S // _L, qpass1, mvec)

            t_in_t = tidx >= tail0
            t_in_m = (tidx >= c0) & (tidx < tail0)
            tvm = plsc.load_gather(q1_v, [jnp.where(t_in_m, tidx - c0, 0)])
            tvt = plsc.load_gather(tl_v, [jnp.where(t_in_t, tidx - tail0, 0)])
            tvec = jnp.where(t_in_t, tvt, jnp.where(t_in_m, tvm, tvec))
            fire(nxt, 1)

            m = lax.reduce_max(mvec, (0,))
            zero = jnp.zeros((_L,), jnp.float32)

            def p2(j, acc):
                accE, accA, accP = acc
                v = s_v[pl.ds(j * _L, _L)]
                e = jnp.exp(v - m)
                sig = 1.0 / (1.0 + jnp.exp(v - tvec))
                return (accE + e, accA + e * sig, accP + e * v * v)
            accE, accA, accP = lax.fori_loop(0, S // _L, p2,
                                             (zero, zero, zero))

            E = lax.reduce_sum(accE, (0,))
            A = lax.reduce_sum(accA, (0,))
            P = lax.reduce_sum(accP, (0,))
            Evec = jnp.full((_L,), E)
            plsc.store_scatter(oA_v, [rvec], jnp.full((_L,), A) / Evec,
                               mask=lane0)
            plsc.store_scatter(oP_v, [rvec], jnp.full((_L,), P) / Evec,
                               mask=lane0)
            return carry

        pltpu.sync_copy(tgt_hbm.at[pl.ds(base, rpt)], tgi_v)
        lax.fori_loop(0, rpt, row_step, 0)
        for k in range(2):
            pltpu.make_async_copy(
                x_hbm.at[base + rpt - 1, pl.ds(offs[k], sizes[k])],
                qbufs[k], qsems[k]).wait()

        pltpu.sync_copy(oA_v, outA_hbm.at[pl.ds(base, rpt)])
        pltpu.sync_copy(oP_v, outP_hbm.at[pl.ds(base, rpt)])

    return sc_partials


def _finish(a, p):
    B = a.shape[0]

    def body(a_ref, p_ref, o_ref):
        o_ref[0, 0] = jnp.mean(-jnp.log(a_ref[...]) + p_ref[...])

    out = pl.pallas_call(
        body,
        out_shape=jax.ShapeDtypeStruct((1, 1), jnp.float32),
        out_specs=pl.BlockSpec(memory_space=pltpu.SMEM),
    )(a.reshape(8, B // 8), p.reshape(8, B // 8))
    return out[0, 0]


def kernel(input, target, samples):
    B, V = input.shape
    S = samples.shape[1]
    tgt = target.astype(jnp.int32)
    smp = samples.astype(jnp.int32)
    tail = input[:, 99968:]
    outA, outP = _make_sc_partials(B, V, S)(input, tail, tgt, smp)
    return _finish(outA, outP)


# speedup vs baseline: 1.0172x; 1.0005x over previous
import functools

import jax
import jax.numpy as jnp
from jax import lax
from jax.experimental import pallas as pl
from jax.experimental.pallas import tpu as pltpu
from jax.experimental.pallas import tpu_sc as plsc

_INFO = plsc.get_sparse_core_info()
_NC, _NS, _L = _INFO.num_cores, _INFO.num_subcores, _INFO.num_lanes
_NW = _NC * _NS


def _make_sc_partials(B, V, S):
    rpt = B // _NW
    sizes = (50048, 49920)
    offs = (0, 50048)
    tail0 = offs[-1] + sizes[-1]
    tailn = V - tail0
    mesh = plsc.VectorSubcoreMesh(core_axis_name="c", subcore_axis_name="s")

    @functools.partial(
        pl.kernel,
        out_type=(
            jax.ShapeDtypeStruct((B,), jnp.float32),
            jax.ShapeDtypeStruct((B,), jnp.float32),
        ),
        mesh=mesh,
        compiler_params=pltpu.CompilerParams(needs_layout_passes=False),
        scratch_types=[
            pltpu.VMEM((rpt,), jnp.int32),
            pltpu.VMEM((S,), jnp.int32),
            pltpu.VMEM((S,), jnp.float32),
            pltpu.VMEM((sizes[0],), jnp.float32),
            pltpu.VMEM((sizes[1],), jnp.float32),
            pltpu.VMEM((tailn,), jnp.float32),
            pltpu.VMEM((rpt,), jnp.float32),
            pltpu.VMEM((rpt,), jnp.float32),
            pltpu.SemaphoreType.DMA,
            pltpu.SemaphoreType.DMA,
        ],
    )
    def sc_partials(x_hbm, tail_hbm, tgt_hbm, smp_hbm, outA_hbm, outP_hbm,
                    tgi_v, sidx_v, s_v, q0_v, q1_v, tl_v,
                    oA_v, oP_v, sem0, sem1):
        wid = lax.axis_index("s") * _NC + lax.axis_index("c")
        base = wid * rpt
        lane0 = lax.iota(jnp.int32, _L) == 0
        qbufs = (q0_v, q1_v)
        qsems = (sem0, sem1)
        neginf = jnp.full((_L,), -jnp.inf, jnp.float32)

        def fire(r, k):
            pltpu.async_copy(x_hbm.at[base + r, pl.ds(offs[k], sizes[k])],
                             qbufs[k], qsems[k])

        for k in range(2):
            fire(0, k)

        def row_step(r, carry):
            pltpu.sync_copy(smp_hbm.at[base + r], sidx_v)
            pltpu.sync_copy(tail_hbm.at[base + r], tl_v)
            rvec = jnp.full((_L,), r, jnp.int32)
            tidx = plsc.load_gather(tgi_v, [rvec])
            nxt = jnp.minimum(r + 1, rpt - 1)

            pltpu.make_async_copy(
                x_hbm.at[base + r, pl.ds(0, sizes[0])], q0_v, sem0).wait()

            def qpass0(j, mv):
                sl = pl.ds(j * _L, _L)
                idx = sidx_v[sl]
                inq = idx < sizes[0]
                v = plsc.load_gather(q0_v, [jnp.where(inq, idx, 0)])
                s_v[sl] = jnp.where(inq, v, s_v[sl])
                return jnp.maximum(mv, jnp.where(inq, v, neginf))
            mvec = lax.fori_loop(0, S // _L, qpass0, neginf)

            t_inq = tidx < sizes[0]
            tv = plsc.load_gather(q0_v, [jnp.where(t_inq, tidx, 0)])
            tvec = jnp.where(t_inq, tv, jnp.zeros((_L,), jnp.float32))
            fire(nxt, 0)

            c0 = offs[1]
            pltpu.make_async_copy(
                x_hbm.at[base + r, pl.ds(c0, sizes[1])], q1_v, sem1).wait()

            def qpass1(j, mv):
                sl = pl.ds(j * _L, _L)
                idx = sidx_v[sl]
                in_t = idx >= tail0
                in_m = (idx >= c0) & (idx < tail0)
                vm = plsc.load_gather(q1_v, [jnp.where(in_m, idx - c0, 0)])
                vt = plsc.load_gather(tl_v, [jnp.where(in_t, idx - tail0, 0)])
                v = jnp.where(in_t, vt, vm)
                inq = idx >= c0
                s_v[sl] = jnp.where(inq, v, s_v[sl])
                return jnp.maximum(mv, jnp.where(inq, v, neginf))
            mvec = lax.fori_loop(0, S // _L, qpass1, mvec)

            t_in_t = tidx >= tail0
            t_in_m = (tidx >= c0) & (tidx < tail0)
            tvm = plsc.load_gather(q1_v, [jnp.where(t_in_m, tidx - c0, 0)])
            tvt = plsc.load_gather(tl_v, [jnp.where(t_in_t, tidx - tail0, 0)])
            tvec = jnp.where(t_in_t, tvt, jnp.where(t_in_m, tvm, tvec))
            fire(nxt, 1)

            m = lax.reduce_max(mvec, (0,))
            zero = jnp.zeros((_L,), jnp.float32)

            def p2(j, acc):
                accE, accA, accP = acc
                v = s_v[pl.ds(j * _L, _L)]
                e = jnp.exp(v - m)
                sig = 1.0 / (1.0 + jnp.exp(v - tvec))
                return (accE + e, accA + e * sig, accP + e * v * v)
            accE, accA, accP = lax.fori_loop(0, S // _L, p2,
                                             (zero, zero, zero))

            E = lax.reduce_sum(accE, (0,))
            A = lax.reduce_sum(accA, (0,))
            P = lax.reduce_sum(accP, (0,))
            Evec = jnp.full((_L,), E)
            plsc.store_scatter(oA_v, [rvec], jnp.full((_L,), A) / Evec,
                               mask=lane0)
            plsc.store_scatter(oP_v, [rvec], jnp.full((_L,), P) / Evec,
                               mask=lane0)
            return carry

        pltpu.sync_copy(tgt_hbm.at[pl.ds(base, rpt)], tgi_v)
        lax.fori_loop(0, rpt, row_step, 0)
        for k in range(2):
            pltpu.make_async_copy(
                x_hbm.at[base + rpt - 1, pl.ds(offs[k], sizes[k])],
                qbufs[k], qsems[k]).wait()

        pltpu.sync_copy(oA_v, outA_hbm.at[pl.ds(base, rpt)])
        pltpu.sync_copy(oP_v, outP_hbm.at[pl.ds(base, rpt)])

    return sc_partials


def _finish(a, p):
    B = a.shape[0]

    def body(a_ref, p_ref, o_ref):
        o_ref[0, 0] = jnp.mean(-jnp.log(a_ref[...]) + p_ref[...])

    out = pl.pallas_call(
        body,
        out_shape=jax.ShapeDtypeStruct((1, 1), jnp.float32),
        out_specs=pl.BlockSpec(memory_space=pltpu.SMEM),
    )(a.reshape(8, B // 8), p.reshape(8, B // 8))
    return out[0, 0]


def kernel(input, target, samples):
    B, V = input.shape
    S = samples.shape[1]
    tgt = target.astype(jnp.int32)
    smp = samples.astype(jnp.int32)
    tail = input[:, 99968:]
    outA, outP = _make_sc_partials(B, V, S)(input, tail, tgt, smp)
    return _finish(outA, outP)
